# baseline (speedup 1.0000x reference)
import functools

import jax
import jax.numpy as jnp
from jax import lax
from jax.experimental import pallas as pl
from jax.experimental.pallas import tpu as pltpu
from jax.experimental.pallas import tpu_sc as plsc

NC = 2
NS = 16
NW = NC * NS
K = 128
NSLOT = 4


@functools.lru_cache(maxsize=None)
def _make_emb(n_t: int, n_bt: int, n_rows: int, d: int):
    n_tp = -(-n_t // 8) * 8
    n_slots = (n_tp // 8) * n_bt * 8
    spw = n_slots // NW
    n_er = d // 8
    mesh = plsc.VectorSubcoreMesh(core_axis_name="c", subcore_axis_name="s")

    @functools.partial(
        pl.kernel,
        mesh=mesh,
        compiler_params=pltpu.CompilerParams(
            use_tc_tiling_on_sc=False, needs_layout_passes=False,
            disable_bounds_checks=True),
        out_type=(
            jax.ShapeDtypeStruct((n_t, n_er, n_bt, 8, K), jnp.float32),
            jax.ShapeDtypeStruct((n_er, 8, K), jnp.float32),
        ),
        scratch_types=[
            pltpu.VMEM((spw, K), jnp.int32),
            pltpu.VMEM((NSLOT, K, d), jnp.float32),
            pltpu.VMEM((NSLOT, n_er, 8, K + 1), jnp.float32),
        ]
        + [pltpu.SemaphoreType.DMA] * (2 * NSLOT),
    )
    def emb(idx_hbm, table_hbm, out_hbm, dump_hbm, idx_v, rows_v, tbuf_v,
            *sems):
        gsems, ssems = sems[:NSLOT], sems[NSLOT:]
        wid = lax.axis_index("s") * NC + lax.axis_index("c")
        s0 = wid * spw
        pltpu.sync_copy(idx_hbm.at[wid], idx_v)

        lane = lax.iota(jnp.int32, 16)
        i_vec = lane & 7
        r_vec = [(lane >> 3) + 2 * h for h in range(d // 16)]
        z16 = lane * 0

        def fire_gather(local, s):
            pltpu.make_async_copy(
                table_hbm.at[idx_v.at[local]], rows_v.at[s], gsems[s]).start()

        for s in range(NSLOT):
            fire_gather(s, s)

        def body(it, carry):
            i = it * NSLOT
            for s in range(NSLOT):
                local = i + s
                sg = s0 + local
                p = sg // (n_bt * 8)
                rem = sg - p * (n_bt * 8)
                c = rem >> 3
                t = p * 8 + (rem & 7)
                pltpu.make_async_copy(
                    table_hbm.at[pl.ds(0, K)], rows_v.at[s], gsems[s]).wait()

                @pl.when(it > 0)
                def _drain():
                    for r in range(n_er):
                        pltpu.make_async_copy(
                            tbuf_v.at[s, r, :, pl.ds(0, K)],
                            dump_hbm.at[r], ssems[s]).wait()

                @plsc.parallel_loop(0, K, 1, unroll=8)
                def _transpose(j):
                    for h in range(d // 16):
                        x = rows_v[s, j, pl.ds(16 * h, 16)]
                        plsc.store_scatter(
                            tbuf_v.at[s], [r_vec[h], i_vec, z16 + j], x)

                @pl.when(t < n_t)
                def _store():
                    for r in range(n_er):
                        pltpu.make_async_copy(
                            tbuf_v.at[s, r, :, pl.ds(0, K)],
                            out_hbm.at[t, r, c], ssems[s]).start()

                @pl.when(t >= n_t)
                def _store_dump():
                    for r in range(n_er):
                        pltpu.make_async_copy(
                            tbuf_v.at[s, r, :, pl.ds(0, K)],
                            dump_hbm.at[r], ssems[s]).start()

                @pl.when(local + NSLOT < spw)
                def _refire():
                    fire_gather(local + NSLOT, s)
            return carry

        lax.fori_loop(0, spw // NSLOT, body, 0)
        for s in range(NSLOT):
            for r in range(n_er):
                pltpu.make_async_copy(
                    tbuf_v.at[s, r, :, pl.ds(0, K)],
                    dump_hbm.at[r], ssems[s]).wait()

    return emb


def kernel(indices, table):
    n_b, n_t = indices.shape
    n_rows, d = table.shape
    assert n_b % K == 0 and d % 16 == 0
    n_bt = n_b // K
    n_tp = -(-n_t // 8) * 8
    n_slots = (n_tp // 8) * n_bt * 8
    assert n_slots % (NW * NSLOT) == 0
    idx_t = indices.astype(jnp.int32).T
    idx_p = jnp.pad(idx_t, ((0, n_tp - n_t), (0, 0)))
    idx_f = idx_p.reshape(n_tp // 8, 8, n_bt, K).transpose(0, 2, 1, 3)
    idx3 = idx_f.reshape(NW, n_slots // NW, K)
    out5, _ = _make_emb(n_t, n_bt, n_rows, d)(idx3, table)
    return out5.transpose(2, 4, 0, 1, 3).reshape(n_b, n_t, d)


# speedup vs baseline: 2.1629x; 2.1629x over previous
import functools

import jax
import jax.numpy as jnp
from jax import lax
from jax.experimental import pallas as pl
from jax.experimental.pallas import tpu as pltpu
from jax.experimental.pallas import tpu_sc as plsc

NC = 2
NS = 16
NW = NC * NS
K = 128
NSLOT = 4


@functools.lru_cache(maxsize=None)
def _make_emb(n_t: int, n_bt: int, n_tab: int, d: int):
    n_chunks = n_t * n_bt
    cpw = n_chunks // NW
    n_er = d // 8
    mesh = plsc.VectorSubcoreMesh(core_axis_name="c", subcore_axis_name="s")

    @functools.partial(
        pl.kernel,
        mesh=mesh,
        compiler_params=pltpu.CompilerParams(
            use_tc_tiling_on_sc=False, needs_layout_passes=False,
            disable_bounds_checks=True),
        out_type=jax.ShapeDtypeStruct((n_t, n_er, n_bt, 8, K), jnp.float32),
        scratch_types=[
            pltpu.VMEM((cpw, K), jnp.int32),
            pltpu.VMEM((NSLOT, K, d), jnp.float32),
            pltpu.VMEM((NSLOT, n_er, 8, K + 1), jnp.float32),
        ]
        + [pltpu.SemaphoreType.DMA] * (2 * NSLOT),
    )
    def emb(idx_hbm, table_hbm, out_hbm, idx_v, rows_v, tbuf_v, *sems):
        gsems, ssems = sems[:NSLOT], sems[NSLOT:]
        wid = lax.axis_index("s") * NC + lax.axis_index("c")
        m0 = wid * cpw
        pltpu.sync_copy(idx_hbm.at[wid], idx_v)

        lane = lax.iota(jnp.int32, 16)
        i_vec = lane & 7
        r_vec = [(lane >> 3) + 2 * h for h in range(d // 16)]
        z16 = lane * 0

        def fire_gather(local, s):
            pltpu.make_async_copy(
                table_hbm.at[idx_v.at[local]], rows_v.at[s], gsems[s]).start()

        for s in range(NSLOT):
            fire_gather(s, s)

        def body(it, carry):
            i = it * NSLOT
            for s in range(NSLOT):
                local = i + s
                m = m0 + local
                t = m // n_bt
                c = m - t * n_bt
                pltpu.make_async_copy(
                    table_hbm.at[pl.ds(0, K)], rows_v.at[s], gsems[s]).wait()

                @pl.when(it > 0)
                def _drain():
                    for r in range(n_er):
                        pltpu.make_async_copy(
                            tbuf_v.at[s, r, :, pl.ds(0, K)],
                            out_hbm.at[0, r, 0], ssems[s]).wait()

                @plsc.parallel_loop(0, K, 1, unroll=8)
                def _transpose(j):
                    for h in range(d // 16):
                        x = rows_v[s, j, pl.ds(16 * h, 16)]
                        plsc.store_scatter(
                            tbuf_v.at[s], [r_vec[h], i_vec, z16 + j], x)

                for r in range(n_er):
                    pltpu.make_async_copy(
                        tbuf_v.at[s, r, :, pl.ds(0, K)],
                        out_hbm.at[t, r, c], ssems[s]).start()

                @pl.when(local + NSLOT < cpw)
                def _refire():
                    fire_gather(local + NSLOT, s)
            return carry

        lax.fori_loop(0, cpw // NSLOT, body, 0)
        for s in range(NSLOT):
            for r in range(n_er):
                pltpu.make_async_copy(
                    tbuf_v.at[s, r, :, pl.ds(0, K)],
                    out_hbm.at[0, r, 0], ssems[s]).wait()

    return emb


def kernel(indices, table):
    n_b, n_t = indices.shape
    n_rows, d = table.shape
    assert n_b % K == 0 and d % 16 == 0
    n_bt = n_b // K
    assert (n_t * n_bt) % (NW * NSLOT) == 0
    pad_w = 128 // d
    table_p = jnp.pad(table, ((0, 0), (0, 128 - d)))
    table_v = table_p.reshape(pad_w * n_rows, d)
    idx_t = (indices.astype(jnp.int32) * pad_w).T.reshape(-1)
    idx3 = idx_t.reshape(NW, (n_t * n_bt) // NW, K)
    out5 = _make_emb(n_t, n_bt, pad_w * n_rows, d)(idx3, table_v)
    return out5.transpose(2, 4, 0, 1, 3).reshape(n_b, n_t, d)
